# speedup vs baseline: 2.7237x; 2.7237x over previous
import functools

import jax
import jax.numpy as jnp
from jax import lax
from jax.experimental import pallas as pl
from jax.experimental.pallas import tpu as pltpu
from jax.experimental.pallas import tpu_sc as plsc

N = 10000
E = 8192
D = 128
ED = 16
G = 64

NC = 2
NS = 16
NW = NC * NS
EPS = E // NS
EPW = E // NW
NPAD = 10240

IC = 128
HCH = EPS // IC
GCH = EPW // IC

EB = 512
NSTEPS = E // EB
KC = 16
MB = 1024


def _sc_call(x, src, dst, batch, ones_c, zeros_n):
    mesh = plsc.VectorSubcoreMesh(core_axis_name="c", subcore_axis_name="s")

    @functools.partial(
        pl.kernel,
        out_type=(
            jax.ShapeDtypeStruct((E, D), jnp.float32),
            jax.ShapeDtypeStruct((E,), jnp.float32),
            jax.ShapeDtypeStruct((E,), jnp.int32),
        ),
        mesh=mesh,
        scratch_types=[
            pltpu.VMEM_SHARED((NPAD,), jnp.float32),
            pltpu.VMEM((HCH, IC), jnp.int32),
            pltpu.VMEM((IC,), jnp.float32),
            pltpu.VMEM((GCH, IC), jnp.int32),
            pltpu.VMEM((GCH, IC), jnp.int32),
            pltpu.VMEM((GCH, IC), jnp.float32),
            pltpu.VMEM((GCH, IC), jnp.float32),
            pltpu.VMEM((GCH, IC), jnp.int32),
            pltpu.VMEM((IC, D), jnp.float32),
            pltpu.SemaphoreType.DMA,
        ],
    )
    def k(x_hbm, src_hbm, dst_hbm, batch_hbm, ones_hbm, zeros_hbm,
          xg_hbm, scale_hbm, ge_hbm,
          cnt_sp, dsth_v, ones_v, idxd_v, idxs_v, cnt_v, scale_v, ge_v,
          rows_v, sem):
        c = lax.axis_index("c")
        s = lax.axis_index("s")
        wid = c * NS + s

        nsl = NPAD // NS
        pltpu.sync_copy(zeros_hbm.at[pl.ds(s * nsl, nsl)],
                        cnt_sp.at[pl.ds(s * nsl, nsl)])
        pltpu.sync_copy(ones_hbm, ones_v)
        plsc.subcore_barrier()

        for j in range(HCH):
            pltpu.sync_copy(dst_hbm.at[pl.ds(s * EPS + j * IC, IC)],
                            dsth_v.at[j])
        for j in range(HCH):
            pltpu.sync_copy(ones_v, cnt_sp.at[dsth_v.at[j]], add=True)
        plsc.subcore_barrier()

        base = wid * EPW
        for j in range(GCH):
            pltpu.sync_copy(dst_hbm.at[pl.ds(base + j * IC, IC)], idxd_v.at[j])
            pltpu.sync_copy(src_hbm.at[pl.ds(base + j * IC, IC)], idxs_v.at[j])
        for j in range(GCH):
            pltpu.async_copy(cnt_sp.at[idxd_v.at[j]], cnt_v.at[j], sem).wait()
            pltpu.async_copy(batch_hbm.at[idxd_v.at[j]], ge_v.at[j], sem).wait()
        for j in range(GCH):
            for i in range(IC // 16):
                cv = cnt_v[j, pl.ds(i * 16, 16)]
                scale_v[j, pl.ds(i * 16, 16)] = 1.0 / jnp.maximum(cv, 1.0)
        for j in range(GCH):
            pltpu.sync_copy(scale_v.at[j], scale_hbm.at[pl.ds(base + j * IC, IC)])
            pltpu.sync_copy(ge_v.at[j], ge_hbm.at[pl.ds(base + j * IC, IC)])
        for j in range(GCH):
            pltpu.async_copy(x_hbm.at[idxs_v.at[j]], rows_v, sem).wait()
            pltpu.sync_copy(rows_v, xg_hbm.at[pl.ds(base + j * IC, IC)])

    return k(x, src, dst, batch, ones_c, zeros_n)


def _mlp_body(ea_ref, w1_ref, b1_ref, w2_ref, b2_ref, w3_ref, b3_ref, h_ref):
    a = jnp.dot(ea_ref[...], w1_ref[...], preferred_element_type=jnp.float32)
    a = jnp.maximum(a + b1_ref[...], 0.0)
    a = jnp.dot(a, w2_ref[...], preferred_element_type=jnp.float32)
    a = jnp.maximum(a + b2_ref[...], 0.0)
    a = jnp.dot(a, w3_ref[...], preferred_element_type=jnp.float32)
    h_ref[...] = jnp.maximum(a + b3_ref[...], 0.0)


def _mlp_call(edge_attr, W1, b1, W2, b2, W3, b3):
    return pl.pallas_call(
        _mlp_body,
        grid=(E // MB,),
        in_specs=[
            pl.BlockSpec((MB, ED), lambda i: (i, 0)),
            pl.BlockSpec((ED, 128), lambda i: (0, 0)),
            pl.BlockSpec((1, 128), lambda i: (0, 0)),
            pl.BlockSpec((128, 256), lambda i: (0, 0)),
            pl.BlockSpec((1, 256), lambda i: (0, 0)),
            pl.BlockSpec((256, 128), lambda i: (0, 0)),
            pl.BlockSpec((1, 128), lambda i: (0, 0)),
        ],
        out_specs=pl.BlockSpec((MB, 128), lambda i: (i, 0)),
        out_shape=jax.ShapeDtypeStruct((E, 128), jnp.float32),
    )(edge_attr, W1, b1.reshape(1, 128), W2, b2.reshape(1, 256),
      W3, b3.reshape(1, 128))


def _main_body(h_ref, xg_ref, scale_ref, ge_ref, w4_ref, b4_ref, out_ref):
    step = pl.program_id(0)
    h = h_ref[...]
    xs = xg_ref[...] * scale_ref[...]
    msg = jnp.dot(xs, b4_ref[...], preferred_element_type=jnp.float32)
    for kc in range(D // KC):
        hc = h[:, kc * KC:(kc + 1) * KC]
        z = (hc[:, :, None] * xs[:, None, :]).reshape(EB, KC * D)
        msg += jnp.dot(z, w4_ref[pl.ds(kc * KC * D, KC * D), :],
                       preferred_element_type=jnp.float32)
    onehot = (ge_ref[...] == lax.broadcasted_iota(jnp.int32, (EB, G), 1))
    onehot = onehot.astype(jnp.float32)
    acc = lax.dot_general(onehot, msg, (((0,), (0,)), ((), ())),
                          preferred_element_type=jnp.float32)

    @pl.when(step == 0)
    def _():
        out_ref[...] = acc

    @pl.when(step != 0)
    def _():
        out_ref[...] += acc


def _main_call(h, xg, scale, ge, W4m, B4):
    return pl.pallas_call(
        _main_body,
        grid=(NSTEPS,),
        in_specs=[
            pl.BlockSpec((EB, D), lambda i: (i, 0)),
            pl.BlockSpec((EB, D), lambda i: (i, 0)),
            pl.BlockSpec((EB, 1), lambda i: (i, 0)),
            pl.BlockSpec((EB, 1), lambda i: (i, 0)),
            pl.BlockSpec((D * D, D), lambda i: (0, 0)),
            pl.BlockSpec((D, D), lambda i: (0, 0)),
        ],
        out_specs=pl.BlockSpec((G, D), lambda i: (0, 0)),
        out_shape=jax.ShapeDtypeStruct((G, D), jnp.float32),
    )(h, xg, scale, ge, W4m, B4)


def kernel(x, edge_index, edge_attr, batch, W1, b1, W2, b2, W3, b3, W4, b4):
    src = edge_index[0]
    dst = edge_index[1]
    ones_c = jnp.ones((IC,), jnp.float32)
    zeros_n = jnp.zeros((NPAD,), jnp.float32)

    xg, scale, ge = _sc_call(x, src, dst, batch, ones_c, zeros_n)
    h = _mlp_call(edge_attr, W1, b1, W2, b2, W3, b3)

    W4m = W4.reshape(D * D, D)
    B4 = b4.reshape(D, D)
    out = _main_call(h, xg, scale.reshape(E, 1), ge.reshape(E, 1), W4m, B4)
    return out


# speedup vs baseline: 3.8631x; 1.4183x over previous
import functools

import jax
import jax.numpy as jnp
from jax import lax
from jax.experimental import pallas as pl
from jax.experimental.pallas import tpu as pltpu
from jax.experimental.pallas import tpu_sc as plsc

N = 10000
E = 8192
D = 128
ED = 16
G = 64

NC = 2
NS = 16
NW = NC * NS
EPS = E // NS
EPW = E // NW
NPAD = 10240

IC = 128
HCH = EPS // IC
GCH = EPW // IC

EB = 512
NSTEPS = E // EB
KC = 16
MB = 1024


def _sc_call(x, src, dst, batch, ones_c, zeros_n):
    mesh = plsc.VectorSubcoreMesh(core_axis_name="c", subcore_axis_name="s")

    @functools.partial(
        pl.kernel,
        out_type=(
            jax.ShapeDtypeStruct((E, D), jnp.float32),
            jax.ShapeDtypeStruct((E,), jnp.float32),
            jax.ShapeDtypeStruct((E,), jnp.int32),
        ),
        mesh=mesh,
        scratch_types=[
            pltpu.VMEM_SHARED((NPAD,), jnp.float32),
            pltpu.VMEM((HCH, IC), jnp.int32),
            pltpu.VMEM((IC,), jnp.float32),
            pltpu.VMEM((GCH, IC), jnp.int32),
            pltpu.VMEM((GCH, IC), jnp.int32),
            pltpu.VMEM((GCH, IC), jnp.float32),
            pltpu.VMEM((GCH, IC), jnp.float32),
            pltpu.VMEM((GCH, IC), jnp.int32),
            pltpu.VMEM((IC, D), jnp.float32),
            pltpu.SemaphoreType.DMA,
        ],
    )
    def k(x_hbm, src_hbm, dst_hbm, batch_hbm, ones_hbm, zeros_hbm,
          xg_hbm, scale_hbm, ge_hbm,
          cnt_sp, dsth_v, ones_v, idxd_v, idxs_v, cnt_v, scale_v, ge_v,
          rows_v, sem):
        c = lax.axis_index("c")
        s = lax.axis_index("s")
        wid = c * NS + s

        nsl = NPAD // NS
        pltpu.sync_copy(zeros_hbm.at[pl.ds(s * nsl, nsl)],
                        cnt_sp.at[pl.ds(s * nsl, nsl)])
        pltpu.sync_copy(ones_hbm, ones_v)
        plsc.subcore_barrier()

        for j in range(HCH):
            pltpu.sync_copy(dst_hbm.at[pl.ds(s * EPS + j * IC, IC)],
                            dsth_v.at[j])
        for j in range(HCH):
            pltpu.sync_copy(ones_v, cnt_sp.at[dsth_v.at[j]], add=True)
        plsc.subcore_barrier()

        base = wid * EPW
        for j in range(GCH):
            pltpu.sync_copy(dst_hbm.at[pl.ds(base + j * IC, IC)], idxd_v.at[j])
            pltpu.sync_copy(src_hbm.at[pl.ds(base + j * IC, IC)], idxs_v.at[j])
        for j in range(GCH):
            pltpu.async_copy(cnt_sp.at[idxd_v.at[j]], cnt_v.at[j], sem).wait()
            pltpu.async_copy(batch_hbm.at[idxd_v.at[j]], ge_v.at[j], sem).wait()
        for j in range(GCH):
            for i in range(IC // 16):
                cv = cnt_v[j, pl.ds(i * 16, 16)]
                scale_v[j, pl.ds(i * 16, 16)] = 1.0 / jnp.maximum(cv, 1.0)
        for j in range(GCH):
            pltpu.sync_copy(scale_v.at[j], scale_hbm.at[pl.ds(base + j * IC, IC)])
            pltpu.sync_copy(ge_v.at[j], ge_hbm.at[pl.ds(base + j * IC, IC)])
        for j in range(GCH):
            pltpu.async_copy(x_hbm.at[idxs_v.at[j]], rows_v, sem).wait()
            pltpu.sync_copy(rows_v, xg_hbm.at[pl.ds(base + j * IC, IC)])

    return k(x, src, dst, batch, ones_c, zeros_n)


def _mlp_body(ea_ref, w1_ref, b1_ref, w2_ref, b2_ref, w3_ref, b3_ref, h_ref):
    a = jnp.dot(ea_ref[...], w1_ref[...], preferred_element_type=jnp.float32)
    a = jnp.maximum(a + b1_ref[...], 0.0)
    a = jnp.dot(a, w2_ref[...], preferred_element_type=jnp.float32)
    a = jnp.maximum(a + b2_ref[...], 0.0)
    a = jnp.dot(a, w3_ref[...], preferred_element_type=jnp.float32)
    h_ref[...] = jnp.maximum(a + b3_ref[...], 0.0)


def _mlp_call(edge_attr, W1, b1, W2, b2, W3, b3):
    return pl.pallas_call(
        _mlp_body,
        grid=(E // MB,),
        in_specs=[
            pl.BlockSpec((MB, ED), lambda i: (i, 0)),
            pl.BlockSpec((ED, 128), lambda i: (0, 0)),
            pl.BlockSpec((1, 128), lambda i: (0, 0)),
            pl.BlockSpec((128, 256), lambda i: (0, 0)),
            pl.BlockSpec((1, 256), lambda i: (0, 0)),
            pl.BlockSpec((256, 128), lambda i: (0, 0)),
            pl.BlockSpec((1, 128), lambda i: (0, 0)),
        ],
        out_specs=pl.BlockSpec((MB, 128), lambda i: (i, 0)),
        out_shape=jax.ShapeDtypeStruct((E, 128), jnp.float32),
    )(edge_attr, W1, b1.reshape(1, 128), W2, b2.reshape(1, 256),
      W3, b3.reshape(1, 128))


def _main_body(h_ref, xg_ref, scale_ref, ge_ref, w4_ref, b4_ref, out_ref):
    step = pl.program_id(0)
    h = h_ref[...]
    xs = xg_ref[...] * scale_ref[...]
    msg = jnp.dot(xs, b4_ref[...], preferred_element_type=jnp.float32)
    h16 = h.astype(jnp.bfloat16)
    xs16 = xs.astype(jnp.bfloat16)
    for kc in range(D // KC):
        hc = h16[:, kc * KC:(kc + 1) * KC]
        z = (hc[:, :, None] * xs16[:, None, :]).reshape(EB, KC * D)
        msg += jnp.dot(z, w4_ref[pl.ds(kc * KC * D, KC * D), :],
                       preferred_element_type=jnp.float32)
    onehot = (ge_ref[...] == lax.broadcasted_iota(jnp.int32, (EB, G), 1))
    onehot = onehot.astype(jnp.float32)
    acc = lax.dot_general(onehot, msg, (((0,), (0,)), ((), ())),
                          preferred_element_type=jnp.float32)

    @pl.when(step == 0)
    def _():
        out_ref[...] = acc

    @pl.when(step != 0)
    def _():
        out_ref[...] += acc


def _main_call(h, xg, scale, ge, W4m, B4):
    return pl.pallas_call(
        _main_body,
        grid=(NSTEPS,),
        in_specs=[
            pl.BlockSpec((EB, D), lambda i: (i, 0)),
            pl.BlockSpec((EB, D), lambda i: (i, 0)),
            pl.BlockSpec((EB, 1), lambda i: (i, 0)),
            pl.BlockSpec((EB, 1), lambda i: (i, 0)),
            pl.BlockSpec((D * D, D), lambda i: (0, 0)),
            pl.BlockSpec((D, D), lambda i: (0, 0)),
        ],
        out_specs=pl.BlockSpec((G, D), lambda i: (0, 0)),
        out_shape=jax.ShapeDtypeStruct((G, D), jnp.float32),
    )(h, xg, scale, ge, W4m, B4)


def kernel(x, edge_index, edge_attr, batch, W1, b1, W2, b2, W3, b3, W4, b4):
    src = edge_index[0]
    dst = edge_index[1]
    ones_c = jnp.ones((IC,), jnp.float32)
    zeros_n = jnp.zeros((NPAD,), jnp.float32)

    xg, scale, ge = _sc_call(x, src, dst, batch, ones_c, zeros_n)
    h = _mlp_call(edge_attr, W1, b1, W2, b2, W3, b3)

    W4m = W4.reshape(D * D, D).astype(jnp.bfloat16)
    B4 = b4.reshape(D, D)
    out = _main_call(h, xg, scale.reshape(E, 1), ge.reshape(E, 1), W4m, B4)
    return out


# speedup vs baseline: 6.8699x; 1.7783x over previous
import functools

import jax
import jax.numpy as jnp
from jax import lax
from jax.experimental import pallas as pl
from jax.experimental.pallas import tpu as pltpu
from jax.experimental.pallas import tpu_sc as plsc

N = 10000
E = 8192
D = 128
ED = 16
G = 64

NC = 2
NS = 16
NW = NC * NS
EPS = E // NS
EPW = E // NW
NPAD = 10240

IC = 128
HCH = EPS // IC
GCH = EPW // IC

EB = 2048
NSTEPS = E // EB
KC = 16
MB = 1024


def _sc_call(x, src, dst, batch, ones_c, zeros_n):
    mesh = plsc.VectorSubcoreMesh(core_axis_name="c", subcore_axis_name="s")

    @functools.partial(
        pl.kernel,
        out_type=(
            jax.ShapeDtypeStruct((E, D), jnp.float32),
            jax.ShapeDtypeStruct((E,), jnp.float32),
            jax.ShapeDtypeStruct((E,), jnp.int32),
        ),
        mesh=mesh,
        scratch_types=[
            pltpu.VMEM_SHARED((NPAD,), jnp.float32),
            pltpu.VMEM((HCH, IC), jnp.int32),
            pltpu.VMEM((IC,), jnp.float32),
            pltpu.VMEM((GCH, IC), jnp.int32),
            pltpu.VMEM((GCH, IC), jnp.int32),
            pltpu.VMEM((GCH, IC), jnp.float32),
            pltpu.VMEM((GCH, IC), jnp.float32),
            pltpu.VMEM((GCH, IC), jnp.int32),
            pltpu.VMEM((IC, D), jnp.float32),
            pltpu.VMEM((IC, D), jnp.float32),
            pltpu.SemaphoreType.DMA,
            pltpu.SemaphoreType.DMA,
        ],
    )
    def k(x_hbm, src_hbm, dst_hbm, batch_hbm, ones_hbm, zeros_hbm,
          xg_hbm, scale_hbm, ge_hbm,
          cnt_sp, dsth_v, ones_v, idxd_v, idxs_v, cnt_v, scale_v, ge_v,
          rows_v0, rows_v1, sem, xsem):
        rows = [rows_v0, rows_v1]
        c = lax.axis_index("c")
        s = lax.axis_index("s")
        wid = c * NS + s

        nsl = NPAD // NS
        pltpu.sync_copy(zeros_hbm.at[pl.ds(s * nsl, nsl)],
                        cnt_sp.at[pl.ds(s * nsl, nsl)])
        pltpu.sync_copy(ones_hbm, ones_v)
        plsc.subcore_barrier()

        for j in range(HCH):
            pltpu.sync_copy(dst_hbm.at[pl.ds(s * EPS + j * IC, IC)],
                            dsth_v.at[j])
        for j in range(HCH):
            pltpu.sync_copy(ones_v, cnt_sp.at[dsth_v.at[j]], add=True)
        plsc.subcore_barrier()

        base = wid * EPW
        for j in range(GCH):
            pltpu.sync_copy(dst_hbm.at[pl.ds(base + j * IC, IC)], idxd_v.at[j])
            pltpu.sync_copy(src_hbm.at[pl.ds(base + j * IC, IC)], idxs_v.at[j])
        xcps = [pltpu.async_copy(x_hbm.at[idxs_v.at[j]], rows[j], xsem)
                for j in range(GCH)]
        for j in range(GCH):
            pltpu.async_copy(cnt_sp.at[idxd_v.at[j]], cnt_v.at[j], sem).wait()
            pltpu.async_copy(batch_hbm.at[idxd_v.at[j]], ge_v.at[j], sem).wait()
        for j in range(GCH):
            for i in range(IC // 16):
                cv = cnt_v[j, pl.ds(i * 16, 16)]
                scale_v[j, pl.ds(i * 16, 16)] = 1.0 / jnp.maximum(cv, 1.0)
        for j in range(GCH):
            pltpu.sync_copy(scale_v.at[j], scale_hbm.at[pl.ds(base + j * IC, IC)])
            pltpu.sync_copy(ge_v.at[j], ge_hbm.at[pl.ds(base + j * IC, IC)])
        for cp in xcps:
            cp.wait()
        for j in range(GCH):
            pltpu.sync_copy(rows[j], xg_hbm.at[pl.ds(base + j * IC, IC)])

    return k(x, src, dst, batch, ones_c, zeros_n)


def _main_body(ea_ref, w1_ref, b1_ref, w2_ref, b2_ref, w3_ref, b3_ref,
               xg_ref, scale_ref, ge_ref, w4t_ref, b4t_ref, out_ref,
               acc_ref):
    step = pl.program_id(0)
    a = jnp.dot(ea_ref[...], w1_ref[...], preferred_element_type=jnp.float32)
    a = jnp.maximum(a + b1_ref[...], 0.0)
    a = jnp.dot(a, w2_ref[...], preferred_element_type=jnp.float32)
    a = jnp.maximum(a + b2_ref[...], 0.0)
    a = jnp.dot(a, w3_ref[...], preferred_element_type=jnp.float32)
    h = jnp.maximum(a + b3_ref[...], 0.0)
    scale_row = scale_ref[...]
    xsT = (xg_ref[...].T * scale_row).astype(jnp.bfloat16)
    hT = h.T.astype(jnp.bfloat16)
    msgT = lax.dot_general(b4t_ref[...], xsT, (((1,), (0,)), ((), ())),
                           preferred_element_type=jnp.float32)
    for kc in range(D // KC):
        pieces = [hT[kc * KC + kk:kc * KC + kk + 1, :] * xsT
                  for kk in range(KC)]
        zT = jnp.concatenate(pieces, axis=0)
        msgT += lax.dot_general(w4t_ref[:, pl.ds(kc * KC * D, KC * D)], zT,
                                (((1,), (0,)), ((), ())),
                                preferred_element_type=jnp.float32)
    onehot = (ge_ref[...] == lax.broadcasted_iota(jnp.int32, (EB, G), 1))
    onehot = onehot.astype(jnp.float32)
    accT = lax.dot_general(msgT, onehot, (((1,), (0,)), ((), ())),
                           preferred_element_type=jnp.float32)

    @pl.when(step == 0)
    def _():
        acc_ref[...] = accT

    @pl.when(step != 0)
    def _():
        acc_ref[...] += accT

    @pl.when(step == NSTEPS - 1)
    def _():
        out_ref[...] = acc_ref[...].T


def _main_call(edge_attr, W1, b1, W2, b2, W3, b3, xg, scale, ge, W4t, B4t):
    return pl.pallas_call(
        _main_body,
        grid=(NSTEPS,),
        in_specs=[
            pl.BlockSpec((EB, ED), lambda i: (i, 0)),
            pl.BlockSpec((ED, 128), lambda i: (0, 0)),
            pl.BlockSpec((1, 128), lambda i: (0, 0)),
            pl.BlockSpec((128, 256), lambda i: (0, 0)),
            pl.BlockSpec((1, 256), lambda i: (0, 0)),
            pl.BlockSpec((256, 128), lambda i: (0, 0)),
            pl.BlockSpec((1, 128), lambda i: (0, 0)),
            pl.BlockSpec((EB, D), lambda i: (i, 0)),
            pl.BlockSpec((1, EB), lambda i: (0, i)),
            pl.BlockSpec((EB, 1), lambda i: (i, 0)),
            pl.BlockSpec((D, D * D), lambda i: (0, 0)),
            pl.BlockSpec((D, D), lambda i: (0, 0)),
        ],
        out_specs=pl.BlockSpec((G, D), lambda i: (0, 0)),
        out_shape=jax.ShapeDtypeStruct((G, D), jnp.float32),
        scratch_shapes=[pltpu.VMEM((D, G), jnp.float32)],
    )(edge_attr, W1, b1.reshape(1, 128), W2, b2.reshape(1, 256),
      W3, b3.reshape(1, 128), xg, scale, ge, W4t, B4t)


def kernel(x, edge_index, edge_attr, batch, W1, b1, W2, b2, W3, b3, W4, b4):
    src = edge_index[0]
    dst = edge_index[1]
    ones_c = jnp.ones((IC,), jnp.float32)
    zeros_n = jnp.zeros((NPAD,), jnp.float32)

    xg, scale, ge = _sc_call(x, src, dst, batch, ones_c, zeros_n)

    W4t = W4.reshape(D * D, D).T.astype(jnp.bfloat16)
    B4t = b4.reshape(D, D).T.astype(jnp.bfloat16)
    out = _main_call(edge_attr, W1, b1, W2, b2, W3, b3,
                     xg, scale.reshape(1, E), ge.reshape(E, 1), W4t, B4t)
    return out
